# baseline (speedup 1.0000x reference)
import functools

import jax
import jax.numpy as jnp
from jax.experimental import pallas as pl
from jax.experimental.pallas import tpu as pltpu

NUM_OPS = 8

BT = 1024
BF = 512


def _route_body(x_ref, o_ref):
    lane = jax.lax.broadcasted_iota(jnp.int32, (1, 128), 1)
    masked = jnp.where(lane < NUM_OPS, x_ref[...], -jnp.inf)
    m = jnp.max(masked)
    idx = jnp.min(jnp.where(masked == m, lane, 128))
    o_ref[0, 0] = idx


def _route(x):
    xs = jax.lax.slice(x, (0, 0, 0), (1, 1, 128)).reshape(1, 128)
    out = pl.pallas_call(
        _route_body,
        out_shape=jax.ShapeDtypeStruct((1, 1), jnp.int32),
        out_specs=pl.BlockSpec(memory_space=pltpu.SMEM),
    )(xs)
    return out.reshape((1,))


def _ffn_body(e_ref, x_ref, w1_ref, b1_ref, w2_ref, b2_ref, o_ref):
    f = pl.program_id(1)
    w1 = w1_ref[0].astype(jnp.bfloat16)
    w2 = w2_ref[0].astype(jnp.bfloat16)
    h = jnp.maximum(
        jnp.dot(x_ref[...], w1, preferred_element_type=jnp.float32)
        + b1_ref[0], 0.0)
    p = jnp.dot(h.astype(jnp.bfloat16), w2, preferred_element_type=jnp.float32)

    @pl.when(f == 0)
    def _():
        o_ref[...] = p + b2_ref[0]

    @pl.when(f > 0)
    def _():
        o_ref[...] += p


@jax.jit
def kernel(x, W1, b1, W2, b2):
    B, S, D = x.shape
    E, _, F = W1.shape
    tokens = B * S
    x2 = x.reshape(tokens, D).astype(jnp.bfloat16)
    b1r = b1.reshape(E, 1, F)
    b2r = b2.reshape(E, 1, D)
    e_idx = _route(x)

    grid = (tokens // BT, F // BF)
    grid_spec = pltpu.PrefetchScalarGridSpec(
        num_scalar_prefetch=1,
        grid=grid,
        in_specs=[
            pl.BlockSpec((BT, D), lambda t, f, e: (t, 0)),
            pl.BlockSpec((1, D, BF), lambda t, f, e: (e[0], 0, f)),
            pl.BlockSpec((1, 1, BF), lambda t, f, e: (e[0], 0, f)),
            pl.BlockSpec((1, BF, D), lambda t, f, e: (e[0], f, 0)),
            pl.BlockSpec((1, 1, D), lambda t, f, e: (e[0], 0, 0)),
        ],
        out_specs=pl.BlockSpec((BT, D), lambda t, f, e: (t, 0)),
    )
    out = pl.pallas_call(
        _ffn_body,
        grid_spec=grid_spec,
        out_shape=jax.ShapeDtypeStruct((tokens, D), jnp.float32),
        compiler_params=pltpu.CompilerParams(
            dimension_semantics=("parallel", "arbitrary"),
        ),
    )(e_idx, x2, W1, b1r, W2, b2r)
    return out.reshape(B, S, D)


# speedup vs baseline: 1.0440x; 1.0440x over previous
import functools

import jax
import jax.numpy as jnp
from jax.experimental import pallas as pl
from jax.experimental.pallas import tpu as pltpu

NUM_OPS = 8

BT = 1024
BF = 512


def _route_body(x_ref, o_ref):
    lane = jax.lax.broadcasted_iota(jnp.int32, (1, 128), 1)
    masked = jnp.where(lane < NUM_OPS, x_ref[...], -jnp.inf)
    m = jnp.max(masked)
    idx = jnp.min(jnp.where(masked == m, lane, 128))
    o_ref[0, 0] = idx


def _route(x):
    xs = jax.lax.slice(x, (0, 0, 0), (1, 1, 128)).reshape(1, 128)
    out = pl.pallas_call(
        _route_body,
        out_shape=jax.ShapeDtypeStruct((1, 1), jnp.int32),
        out_specs=pl.BlockSpec(memory_space=pltpu.SMEM),
    )(xs)
    return out.reshape((1,))


def _ffn_body(e_ref, x_ref, w1_ref, b1_ref, w2_ref, b2_ref, o_ref):
    f = pl.program_id(1)
    h = jnp.maximum(
        jnp.dot(x_ref[...], w1_ref[0], preferred_element_type=jnp.float32)
        + b1_ref[0], 0.0)
    p = jnp.dot(h, w2_ref[0], preferred_element_type=jnp.float32)

    @pl.when(f == 0)
    def _():
        o_ref[...] = p + b2_ref[0]

    @pl.when(f > 0)
    def _():
        o_ref[...] += p


@jax.jit
def kernel(x, W1, b1, W2, b2):
    B, S, D = x.shape
    E, _, F = W1.shape
    tokens = B * S
    x2 = x.reshape(tokens, D)
    b1r = b1.reshape(E, 1, F)
    b2r = b2.reshape(E, 1, D)
    e_idx = _route(x)

    grid = (tokens // BT, F // BF)
    grid_spec = pltpu.PrefetchScalarGridSpec(
        num_scalar_prefetch=1,
        grid=grid,
        in_specs=[
            pl.BlockSpec((BT, D), lambda t, f, e: (t, 0)),
            pl.BlockSpec((1, D, BF), lambda t, f, e: (e[0], 0, f)),
            pl.BlockSpec((1, 1, BF), lambda t, f, e: (e[0], 0, f)),
            pl.BlockSpec((1, BF, D), lambda t, f, e: (e[0], f, 0)),
            pl.BlockSpec((1, 1, D), lambda t, f, e: (e[0], 0, 0)),
        ],
        out_specs=pl.BlockSpec((BT, D), lambda t, f, e: (t, 0)),
    )
    out = pl.pallas_call(
        _ffn_body,
        grid_spec=grid_spec,
        out_shape=jax.ShapeDtypeStruct((tokens, D), jnp.float32),
        compiler_params=pltpu.CompilerParams(
            dimension_semantics=("parallel", "arbitrary"),
            vmem_limit_bytes=100 * 1024 * 1024,
        ),
    )(e_idx, x2, W1, b1r, W2, b2r)
    return out.reshape(B, S, D)


# speedup vs baseline: 1.0463x; 1.0022x over previous
import functools

import jax
import jax.numpy as jnp
from jax.experimental import pallas as pl
from jax.experimental.pallas import tpu as pltpu

NUM_OPS = 8

BT1 = 2048
BF1 = 512
BT2 = 1024
BK2 = 1024


def _route_body(x_ref, o_ref):
    lane = jax.lax.broadcasted_iota(jnp.int32, (1, 128), 1)
    masked = jnp.where(lane < NUM_OPS, x_ref[...], -jnp.inf)
    m = jnp.max(masked)
    idx = jnp.min(jnp.where(masked == m, lane, 128))
    o_ref[0, 0] = idx


def _route(x):
    xs = jax.lax.slice(x, (0, 0, 0), (1, 1, 128)).reshape(1, 128)
    out = pl.pallas_call(
        _route_body,
        out_shape=jax.ShapeDtypeStruct((1, 1), jnp.int32),
        out_specs=pl.BlockSpec(memory_space=pltpu.SMEM),
    )(xs)
    return out.reshape((1,))


def _h_body(e_ref, x_ref, w1_ref, b1_ref, h_ref):
    w1 = w1_ref[0].astype(jnp.bfloat16)
    hp = jnp.dot(x_ref[...], w1, preferred_element_type=jnp.float32) + b1_ref[0]
    h_ref[...] = jnp.maximum(hp, 0.0).astype(jnp.bfloat16)


def _out_body(e_ref, h_ref, w2_ref, b2_ref, o_ref):
    k = pl.program_id(1)
    w2 = w2_ref[0].astype(jnp.bfloat16)
    p = jnp.dot(h_ref[...], w2, preferred_element_type=jnp.float32)

    @pl.when(k == 0)
    def _():
        o_ref[...] = p + b2_ref[0]

    @pl.when(k > 0)
    def _():
        o_ref[...] += p


@jax.jit
def kernel(x, W1, b1, W2, b2):
    B, S, D = x.shape
    E, _, F = W1.shape
    tokens = B * S
    x2 = x.reshape(tokens, D).astype(jnp.bfloat16)
    b1r = b1.reshape(E, 1, F)
    b2r = b2.reshape(E, 1, D)
    e_idx = _route(x)

    h = pl.pallas_call(
        _h_body,
        grid_spec=pltpu.PrefetchScalarGridSpec(
            num_scalar_prefetch=1,
            grid=(tokens // BT1, F // BF1),
            in_specs=[
                pl.BlockSpec((BT1, D), lambda t, f, e: (t, 0)),
                pl.BlockSpec((1, D, BF1), lambda t, f, e: (e[0], 0, f)),
                pl.BlockSpec((1, 1, BF1), lambda t, f, e: (e[0], 0, f)),
            ],
            out_specs=pl.BlockSpec((BT1, BF1), lambda t, f, e: (t, f)),
        ),
        out_shape=jax.ShapeDtypeStruct((tokens, F), jnp.bfloat16),
        compiler_params=pltpu.CompilerParams(
            dimension_semantics=("parallel", "parallel"),
            vmem_limit_bytes=63 * 1024 * 1024,
        ),
    )(e_idx, x2, W1, b1r)

    out = pl.pallas_call(
        _out_body,
        grid_spec=pltpu.PrefetchScalarGridSpec(
            num_scalar_prefetch=1,
            grid=(tokens // BT2, F // BK2),
            in_specs=[
                pl.BlockSpec((BT2, BK2), lambda t, k, e: (t, k)),
                pl.BlockSpec((1, BK2, D), lambda t, k, e: (e[0], k, 0)),
                pl.BlockSpec((1, 1, D), lambda t, k, e: (e[0], 0, 0)),
            ],
            out_specs=pl.BlockSpec((BT2, D), lambda t, k, e: (t, 0)),
        ),
        out_shape=jax.ShapeDtypeStruct((tokens, D), jnp.float32),
        compiler_params=pltpu.CompilerParams(
            dimension_semantics=("parallel", "arbitrary"),
            vmem_limit_bytes=63 * 1024 * 1024,
        ),
    )(e_idx, h, W2, b2r)
    return out.reshape(B, S, D)
